# baseline (speedup 1.0000x reference)
import jax
import jax.numpy as jnp
from jax.experimental import pallas as pl
from jax.experimental.pallas import tpu as pltpu

_NC = 8
_SUB = 8


def _row_softmax_kernel(*args):
    x_refs = args[:_NC]
    o_hbm = args[_NC]
    bufs = args[_NC + 1:_NC + 3]
    sems = args[_NC + 3]

    i = pl.program_id(0)
    n = pl.num_programs(0)

    xs = [r[...] for r in x_refs]
    m = jnp.max(jnp.stack([jnp.max(x) for x in xs]))
    es = [jnp.exp(x - m) for x in xs]
    s = jnp.sum(jnp.stack([jnp.sum(e) for e in es]))
    inv = 1.0 / s

    def chunk_copy(buf, slot, k):
        return pltpu.make_async_copy(
            buf.at[pl.ds(k * _SUB, _SUB), :],
            o_hbm.at[pl.ds((i * _NC + k) * _SUB, _SUB), :],
            sems.at[slot, k],
        )

    def do_slot(buf, slot):
        @pl.when(i >= 2)
        def _():
            for k in range(_NC):
                chunk_copy(buf, slot, k).wait()

        for k in range(_NC):
            buf[pl.ds(k * _SUB, _SUB), :] = es[k] * inv
        for k in range(_NC):
            chunk_copy(buf, slot, k).start()

    @pl.when(i % 2 == 0)
    def _():
        do_slot(bufs[0], 0)

    @pl.when(i % 2 == 1)
    def _():
        do_slot(bufs[1], 1)

    @pl.when(i == n - 1)
    def _():
        for slot in range(2):
            for k in range(_NC):
                chunk_copy(bufs[slot], slot, k).wait()


def kernel(logits):
    n_rows, n_cols = logits.shape
    lanes = n_cols // (_NC * _SUB)
    x = logits.reshape(n_rows * _NC * _SUB, lanes)

    in_specs = [
        pl.BlockSpec((_SUB, lanes), lambda i, _k=k: (i * _NC + _k, 0))
        for k in range(_NC)
    ]
    out = pl.pallas_call(
        _row_softmax_kernel,
        grid=(n_rows,),
        in_specs=in_specs,
        out_specs=pl.BlockSpec(memory_space=pl.ANY),
        out_shape=jax.ShapeDtypeStruct(x.shape, x.dtype),
        scratch_shapes=[
            pltpu.VMEM((_NC * _SUB, lanes), jnp.float32),
            pltpu.VMEM((_NC * _SUB, lanes), jnp.float32),
            pltpu.SemaphoreType.DMA((2, _NC)),
        ],
    )(*([x] * _NC))
    return out.reshape(n_rows, n_cols)
